# baseline (speedup 1.0000x reference)
import functools

import jax
import jax.numpy as jnp
from jax import lax
from jax.experimental import pallas as pl
from jax.experimental.pallas import tpu as pltpu
from jax.experimental.pallas import tpu_sc as plsc

_NC = 2
_NS = 16
_NW = _NC * _NS
_CH = 112
_K = 4


def _sc_sum_pool(x2, emb, b_total, seq_pad):
    d = emb.shape[1]
    bpw = b_total // _NW
    cpr = seq_pad // _CH
    tot = bpw * cpr
    mesh = plsc.VectorSubcoreMesh(core_axis_name="c", subcore_axis_name="s")

    @functools.partial(
        pl.kernel,
        out_type=jax.ShapeDtypeStruct((b_total, d), jnp.float32),
        mesh=mesh,
        compiler_params=pltpu.CompilerParams(use_tc_tiling_on_sc=False),
        scratch_types=(
            [pltpu.VMEM((tot, _CH), jnp.int32),
             pltpu.VMEM((_K, _CH), jnp.int32)]
            + [pltpu.VMEM((_CH, d), jnp.float32) for _ in range(_K)]
            + [pltpu.VMEM((bpw, d), jnp.float32)]
            + [pltpu.VMEM_SHARED((_NS * bpw, d), jnp.float32)]
            + [pltpu.SemaphoreType.DMA for _ in range(_K)]
        ),
    )
    def pool(x_hbm, emb_hbm, out_hbm, idx_v, dst_v, *refs):
        bufs = refs[:_K]
        zbuf = refs[_K]
        acc = refs[_K + 1]
        sems = refs[_K + 2:]

        s = lax.axis_index("s")
        c = lax.axis_index("c")
        wid = s * _NC + c
        pltpu.sync_copy(x_hbm.at[pl.ds(wid * tot, tot)], idx_v)
        arow = s * bpw

        zv = jnp.zeros((16,), jnp.float32)

        def zrow(i, carry):
            for w in range(d // 16):
                zbuf[i, pl.ds(w * 16, 16)] = zv
            return carry

        lax.fori_loop(0, bpw, zrow, jnp.int32(0))
        pltpu.sync_copy(zbuf, acc.at[pl.ds(arow, bpw)])

        def issue(g, k):
            return pltpu.async_copy(emb_hbm.at[idx_v.at[g]], bufs[k],
                                    sems[k])

        def drain(k):
            pltpu.make_async_copy(emb_hbm.at[idx_v.at[0]], bufs[k],
                                  sems[k]).wait()

        for k in range(_K):
            issue(k, k)

        def body(i, carry):
            for k in range(_K):
                g = i * _K + k
                rv = jnp.full((16,), arow + g // cpr, jnp.int32)
                for w in range(_CH // 16):
                    dst_v[k, pl.ds(w * 16, 16)] = rv
                drain(k)
                pltpu.sync_copy(bufs[k], acc.at[dst_v.at[k]], add=True)
                issue(jnp.minimum(g + _K, tot - 1), k)
            return carry

        lax.fori_loop(0, tot // _K, body, jnp.int32(0))
        for k in range(_K):
            drain(k)
        pltpu.sync_copy(acc.at[pl.ds(arow, bpw)],
                        out_hbm.at[pl.ds(wid * bpw, bpw)])

    return pool(x2, emb)


def _tc_mlp(sums, x, row0, W1, b1, W2p, b2p, seq_pad):
    b_total, _ = sums.shape
    h = W1.shape[1]
    o = W2p.shape[1]

    def body(s_ref, x_ref, r0_ref, w1_ref, b1_ref, w2_ref, b2_ref, o_ref):
        lenf = jnp.sum((x_ref[...] != 0).astype(jnp.float32), axis=1,
                       keepdims=True)
        pad_cnt = seq_pad - lenf
        avg = (s_ref[...] - pad_cnt * r0_ref[...]) / jnp.maximum(lenf, 1.0)
        hh = jnp.dot(avg, w1_ref[...], preferred_element_type=jnp.float32)
        hh = jnp.maximum(hh + b1_ref[...], 0.0)
        o_ref[...] = jnp.dot(hh, w2_ref[...],
                             preferred_element_type=jnp.float32) + b2_ref[...]

    return pl.pallas_call(
        body,
        out_shape=jax.ShapeDtypeStruct((b_total, o), jnp.float32),
    )(sums, x, row0, W1, b1.reshape(1, h), W2p, b2p.reshape(1, o))


def kernel(x, emb_table, W1, b1, W2, b2):
    x = x.astype(jnp.int32)
    b_total, s = x.shape
    cpr = -(-s // _CH)
    seq_pad = cpr * _CH
    x2 = jnp.pad(x, ((0, 0), (0, seq_pad - s))).reshape(-1, _CH)
    sums = _sc_sum_pool(x2, emb_table, b_total, seq_pad)
    o = 128
    w2p = jnp.pad(W2, ((0, 0), (0, o - W2.shape[1])))
    b2p = jnp.pad(b2, (0, o - b2.shape[0]))
    row0 = emb_table[0:1]
    out = _tc_mlp(sums, x, row0, W1, b1, w2p, b2p, float(seq_pad))
    return out[:, : W2.shape[1]]
